# Initial kernel scaffold; baseline (speedup 1.0000x reference)
import jax
import jax.numpy as jnp
from jax.experimental import pallas as pl


def kernel(x, edge_index, pathway_idx, W1, b1, W2, b2, Wih, Whh, bih, bhh, Wl1, bl1, Wl2, bl2):
    raise NotImplementedError("write your pallas kernel here")


# speedup vs baseline: 26.3641x; 26.3641x over previous
import functools

import jax
import jax.numpy as jnp
from jax import lax
from jax.experimental import pallas as pl
from jax.experimental.pallas import tpu as pltpu
from jax.experimental.pallas import tpu_sc as plsc

_NC = 2
_NS = 16
_NW = _NC * _NS
_SUB = 128
_K = 16
_F = 8


def _edge_agg_call(n_pad, total_rows):
  rows_w = total_rows // _NW
  outer = rows_w // _K
  zrows = n_pad // _NS
  mesh = plsc.VectorSubcoreMesh(core_axis_name="c", subcore_axis_name="s")

  @functools.partial(
      pl.kernel,
      out_type=jax.ShapeDtypeStruct((_NC, n_pad, _F), jnp.float32),
      mesh=mesh,
      scratch_types=[
          pltpu.VMEM((_K, _SUB), jnp.int32),
          pltpu.VMEM((_K, _SUB), jnp.int32),
          pltpu.VMEM((_K * _SUB, _F), jnp.float32),
          pltpu.VMEM_SHARED((n_pad, _F), jnp.float32),
          pltpu.SemaphoreType.DMA,
      ],
      compiler_params=pltpu.CompilerParams(use_tc_tiling_on_sc=False),
  )
  def k(table_hbm, zero_hbm, src_hbm, dst_hbm, out_hbm,
        idx_s, idx_d, rows, agg_sp, sem):
    c = lax.axis_index("c")
    s = lax.axis_index("s")
    wid = c * _NS + s
    pltpu.sync_copy(zero_hbm.at[pl.ds(s * zrows, zrows)],
                    agg_sp.at[pl.ds(s * zrows, zrows)])
    plsc.subcore_barrier()

    row0 = wid * rows_w

    def body(i, carry):
      base = row0 + i * _K
      pltpu.sync_copy(src_hbm.at[pl.ds(base, _K)], idx_s)
      pltpu.sync_copy(dst_hbm.at[pl.ds(base, _K)], idx_d)
      cps = []
      for j in range(_K):
        cps.append(pltpu.async_copy(
            table_hbm.at[idx_s.at[j]],
            rows.at[pl.ds(j * _SUB, _SUB)], sem))
      for cp in cps:
        cp.wait()
      for j in range(_K):
        pltpu.sync_copy(rows.at[pl.ds(j * _SUB, _SUB)],
                        agg_sp.at[idx_d.at[j]], add=True)
      return carry

    lax.fori_loop(0, outer, body, 0)
    plsc.subcore_barrier()
    pltpu.sync_copy(agg_sp.at[pl.ds(s * zrows, zrows)],
                    out_hbm.at[c, pl.ds(s * zrows, zrows)])

  return k


def _gather_rows_call(n_rows_out):
  rows_w = n_rows_out // (_NW * _SUB)
  per_w = rows_w * _SUB
  mesh = plsc.VectorSubcoreMesh(core_axis_name="c", subcore_axis_name="s")

  @functools.partial(
      pl.kernel,
      out_type=jax.ShapeDtypeStruct((n_rows_out, _F), jnp.float32),
      mesh=mesh,
      scratch_types=[
          pltpu.VMEM((rows_w, _SUB), jnp.int32),
          pltpu.VMEM((per_w, _F), jnp.float32),
          pltpu.SemaphoreType.DMA,
      ],
      compiler_params=pltpu.CompilerParams(use_tc_tiling_on_sc=False),
  )
  def k(table_hbm, idx_hbm, out_hbm, idx_v, rows, sem):
    c = lax.axis_index("c")
    s = lax.axis_index("s")
    wid = c * _NS + s
    pltpu.sync_copy(idx_hbm.at[pl.ds(wid * rows_w, rows_w)], idx_v)
    cps = []
    for j in range(rows_w):
      cps.append(pltpu.async_copy(
          table_hbm.at[idx_v.at[j]],
          rows.at[pl.ds(j * _SUB, _SUB)], sem))
    for cp in cps:
      cp.wait()
    pltpu.sync_copy(rows, out_hbm.at[pl.ds(wid * per_w, per_w)])

  return k


def _dense_body(x_ref, p0_ref, p1_ref, w_ref, b_ref, o_ref):
  a = x_ref[...] + p0_ref[...] + p1_ref[...]
  o_ref[...] = jnp.maximum(
      jnp.dot(a, w_ref[...], preferred_element_type=jnp.float32) + b_ref[...],
      0.0)


def _dense(x, p0, p1, wt, brow, blk):
  n = x.shape[0]
  return pl.pallas_call(
      _dense_body,
      grid=(n // blk,),
      in_specs=[
          pl.BlockSpec((blk, _F), lambda i: (i, 0)),
          pl.BlockSpec((blk, _F), lambda i: (i, 0)),
          pl.BlockSpec((blk, _F), lambda i: (i, 0)),
          pl.BlockSpec((_F, _F), lambda i: (0, 0)),
          pl.BlockSpec((1, _F), lambda i: (0, 0)),
      ],
      out_specs=pl.BlockSpec((blk, _F), lambda i: (i, 0)),
      out_shape=jax.ShapeDtypeStruct((n, _F), jnp.float32),
  )(x, p0, p1, wt, brow)


def _head_body(f0_ref, f1_ref, f2_ref, wih_ref, whh_ref, bih_ref, bhh_ref,
               wl1_ref, bl1_ref, wl2_ref, bl2_ref, o_ref):
  p = f0_ref.shape[0]
  d = 3
  f0 = f0_ref[...]
  f1 = f1_ref[...]
  f2 = f2_ref[...]
  wih_t = wih_ref[...]
  whh_t = whh_ref[...]
  bsum = bih_ref[...] + bhh_ref[...]
  q_star = jnp.zeros((p, 2 * d), jnp.float32)
  hs = jnp.zeros((p, d), jnp.float32)
  cs = jnp.zeros((p, d), jnp.float32)
  for _ in range(2):
    gates = (jnp.dot(q_star, wih_t, preferred_element_type=jnp.float32)
             + jnp.dot(hs, whh_t, preferred_element_type=jnp.float32) + bsum)
    gi = gates[:, 0:d]
    gf = gates[:, d:2 * d]
    gg = gates[:, 2 * d:3 * d]
    go = gates[:, 3 * d:4 * d]
    cs = jax.nn.sigmoid(gf) * cs + jax.nn.sigmoid(gi) * jnp.tanh(gg)
    hs = jax.nn.sigmoid(go) * jnp.tanh(cs)
    q = hs
    e = f0 * q[:, 0:1] + f1 * q[:, 1:2] + f2 * q[:, 2:3]
    m = jnp.max(e, axis=1, keepdims=True)
    ex = jnp.exp(e - m)
    alpha = ex / jnp.sum(ex, axis=1, keepdims=True)
    r0 = jnp.sum(alpha * f0, axis=1, keepdims=True)
    r1 = jnp.sum(alpha * f1, axis=1, keepdims=True)
    r2 = jnp.sum(alpha * f2, axis=1, keepdims=True)
    q_star = jnp.concatenate([q, r0, r1, r2], axis=1)
  xx = bl1_ref[...]
  for dd in range(2 * d):
    xx = xx + jnp.dot(wl1_ref[dd], q_star[:, dd:dd + 1],
                      preferred_element_type=jnp.float32)
  xx = jnp.maximum(xx, 0.0)
  o_ref[...] = jax.nn.sigmoid(
      jnp.dot(wl2_ref[...], xx, preferred_element_type=jnp.float32)
      + bl2_ref[...])


def _pad_mat(w, rows, cols):
  return jnp.zeros((rows, cols), jnp.float32).at[:w.shape[0], :w.shape[1]].set(w)


def kernel(x, edge_index, pathway_idx, W1, b1, W2, b2,
           Wih, Whh, bih, bhh, Wl1, bl1, Wl2, bl2):
  n, d = x.shape
  e = edge_index.shape[1]
  p, l = pathway_idx.shape

  unit = _NW * _K * _SUB
  e_pad = ((e + unit - 1) // unit) * unit
  total_rows = e_pad // _SUB
  n_pad = ((n + _NS * 8 - 1) // (_NS * 8)) * (_NS * 8)
  if n_pad == n:
    n_pad += _NS * 8
  src = jnp.concatenate(
      [edge_index[0], jnp.zeros((e_pad - e,), jnp.int32)]).reshape(
          total_rows, _SUB)
  dst = jnp.concatenate(
      [edge_index[1], jnp.full((e_pad - e,), n, jnp.int32)]).reshape(
          total_rows, _SUB)

  zeros8 = jnp.zeros((n_pad, _F), jnp.float32)
  x8 = jnp.pad(x, ((0, 0), (0, _F - d)))

  part1 = _edge_agg_call(n_pad, total_rows)(x8, zeros8, src, dst)
  h1 = _dense(x8, part1[0, :n], part1[1, :n],
              _pad_mat(W1.T, _F, _F), _pad_mat(b1.reshape(1, -1), 1, _F), 2000)
  part2 = _edge_agg_call(n_pad, total_rows)(h1, zeros8, src, dst)
  h2 = _dense(h1, part2[0, :n], part2[1, :n],
              _pad_mat(W2.T, _F, _F), _pad_mat(b2.reshape(1, -1), 1, _F), 2000)

  pl_flat = p * l
  g_unit = _NW * _SUB
  g_pad = ((pl_flat + g_unit - 1) // g_unit) * g_unit
  pidx = jnp.concatenate(
      [pathway_idx.reshape(-1), jnp.zeros((g_pad - pl_flat,), jnp.int32)]
  ).reshape(g_pad // _SUB, _SUB)
  feat = _gather_rows_call(g_pad)(h2, pidx)
  feat = feat[:pl_flat].reshape(p, l, _F)
  f0 = feat[:, :, 0]
  f1 = feat[:, :, 1]
  f2 = feat[:, :, 2]

  wl1_stack = jnp.transpose(Wl1.reshape(p, p, 2 * d), (2, 0, 1))
  res = pl.pallas_call(
      _head_body,
      in_specs=[
          pl.BlockSpec((p, l), lambda: (0, 0)),
          pl.BlockSpec((p, l), lambda: (0, 0)),
          pl.BlockSpec((p, l), lambda: (0, 0)),
          pl.BlockSpec((2 * d, 4 * d), lambda: (0, 0)),
          pl.BlockSpec((d, 4 * d), lambda: (0, 0)),
          pl.BlockSpec((1, 4 * d), lambda: (0, 0)),
          pl.BlockSpec((1, 4 * d), lambda: (0, 0)),
          pl.BlockSpec((2 * d, p, p), lambda: (0, 0, 0)),
          pl.BlockSpec((p, 1), lambda: (0, 0)),
          pl.BlockSpec((1, p), lambda: (0, 0)),
          pl.BlockSpec((1, 1), lambda: (0, 0)),
      ],
      out_specs=pl.BlockSpec((1, 1), lambda: (0, 0)),
      out_shape=jax.ShapeDtypeStruct((1, 1), jnp.float32),
  )(f0, f1, f2, Wih.T, Whh.T, bih.reshape(1, 4 * d), bhh.reshape(1, 4 * d),
    wl1_stack, bl1.reshape(p, 1), Wl2, bl2.reshape(1, 1))
  return res.reshape(1)


# speedup vs baseline: 35.9634x; 1.3641x over previous
import functools

import jax
import jax.numpy as jnp
from jax import lax
from jax.experimental import pallas as pl
from jax.experimental.pallas import tpu as pltpu
from jax.experimental.pallas import tpu_sc as plsc

_NC = 2
_NS = 16
_NW = _NC * _NS
_SUB = 128
_K = 8
_F = 8


def _edge_agg_call(n, total_rows):
  zrows = n // _NS
  total_chunks = total_rows // _K
  nch_lo = total_chunks // _NW
  n_hi = total_chunks - nch_lo * _NW
  mesh = plsc.VectorSubcoreMesh(core_axis_name="c", subcore_axis_name="s")

  @functools.partial(
      pl.kernel,
      out_type=jax.ShapeDtypeStruct((_NC, n, _F), jnp.float32),
      mesh=mesh,
      scratch_types=[
          pltpu.VMEM((_K, _SUB), jnp.int32),
          pltpu.VMEM((_K, _SUB), jnp.int32),
          pltpu.VMEM((_K * _SUB, _F), jnp.float32),
          pltpu.VMEM_SHARED((n, _F), jnp.float32),
          pltpu.SemaphoreType.DMA,
      ],
      compiler_params=pltpu.CompilerParams(use_tc_tiling_on_sc=False),
  )
  def k(table_hbm, zero_hbm, ei_hbm, out_hbm, idx_s, idx_d, rows, agg_sp, sem):
    c = lax.axis_index("c")
    s = lax.axis_index("s")
    wid = c * _NS + s
    pltpu.sync_copy(zero_hbm.at[pl.ds(s * zrows, zrows)],
                    agg_sp.at[pl.ds(s * zrows, zrows)])
    plsc.subcore_barrier()

    base_chunk = wid * nch_lo + jnp.minimum(wid, n_hi)
    nch = nch_lo + jnp.where(wid < n_hi, 1, 0)

    def body(i, carry):
      base = (base_chunk + i) * _K
      pltpu.sync_copy(ei_hbm.at[0, pl.ds(base, _K)], idx_s)
      pltpu.sync_copy(ei_hbm.at[1, pl.ds(base, _K)], idx_d)
      cps = []
      for j in range(_K):
        cps.append(pltpu.async_copy(
            table_hbm.at[idx_s.at[j]],
            rows.at[pl.ds(j * _SUB, _SUB)], sem))
      for cp in cps:
        cp.wait()
      for j in range(_K):
        pltpu.sync_copy(rows.at[pl.ds(j * _SUB, _SUB)],
                        agg_sp.at[idx_d.at[j]], add=True)
      return carry

    lax.fori_loop(0, nch, body, 0)
    plsc.subcore_barrier()
    pltpu.sync_copy(agg_sp.at[pl.ds(s * zrows, zrows)],
                    out_hbm.at[c, pl.ds(s * zrows, zrows)])

  return k


def _gather_rows_call(n_rows_out):
  rows_w = n_rows_out // (_NW * _SUB)
  per_w = rows_w * _SUB
  mesh = plsc.VectorSubcoreMesh(core_axis_name="c", subcore_axis_name="s")

  @functools.partial(
      pl.kernel,
      out_type=jax.ShapeDtypeStruct((n_rows_out, _F), jnp.float32),
      mesh=mesh,
      scratch_types=[
          pltpu.VMEM((rows_w, _SUB), jnp.int32),
          pltpu.VMEM((per_w, _F), jnp.float32),
          pltpu.SemaphoreType.DMA,
      ],
      compiler_params=pltpu.CompilerParams(use_tc_tiling_on_sc=False),
  )
  def k(table_hbm, idx_hbm, out_hbm, idx_v, rows, sem):
    c = lax.axis_index("c")
    s = lax.axis_index("s")
    wid = c * _NS + s
    pltpu.sync_copy(idx_hbm.at[pl.ds(wid * rows_w, rows_w)], idx_v)
    cps = []
    for j in range(rows_w):
      cps.append(pltpu.async_copy(
          table_hbm.at[idx_v.at[j]],
          rows.at[pl.ds(j * _SUB, _SUB)], sem))
    for cp in cps:
      cp.wait()
    pltpu.sync_copy(rows, out_hbm.at[pl.ds(wid * per_w, per_w)])

  return k


def _dense_body(x_ref, p_ref, w_ref, b_ref, o_ref):
  a = x_ref[...] + p_ref[0] + p_ref[1]
  o_ref[...] = jnp.maximum(
      jnp.dot(a, w_ref[...], preferred_element_type=jnp.float32) + b_ref[...],
      0.0)


def _dense(xr, pr, wbig, bbig, blk):
  nr = xr.shape[0]
  del blk
  return pl.pallas_call(
      _dense_body,
      in_specs=[
          pl.BlockSpec((nr, 128), lambda: (0, 0)),
          pl.BlockSpec((2, nr, 128), lambda: (0, 0, 0)),
          pl.BlockSpec((128, 128), lambda: (0, 0)),
          pl.BlockSpec((1, 128), lambda: (0, 0)),
      ],
      out_specs=pl.BlockSpec((nr, 128), lambda: (0, 0)),
      out_shape=jax.ShapeDtypeStruct((nr, 128), jnp.float32),
  )(xr, pr, wbig, bbig)


def _head_body(f0_ref, f1_ref, f2_ref, wih_ref, whh_ref, bih_ref, bhh_ref,
               wl1_ref, bl1_ref, wl2_ref, bl2_ref, o_ref):
  p = f0_ref.shape[0]
  d = 3
  f0 = f0_ref[...]
  f1 = f1_ref[...]
  f2 = f2_ref[...]
  wih_t = wih_ref[...]
  whh_t = whh_ref[...]
  bsum = bih_ref[...] + bhh_ref[...]
  q_star = jnp.zeros((p, 2 * d), jnp.float32)
  hs = jnp.zeros((p, d), jnp.float32)
  cs = jnp.zeros((p, d), jnp.float32)
  for _ in range(2):
    gates = (jnp.dot(q_star, wih_t, preferred_element_type=jnp.float32)
             + jnp.dot(hs, whh_t, preferred_element_type=jnp.float32) + bsum)
    gi = gates[:, 0:d]
    gf = gates[:, d:2 * d]
    gg = gates[:, 2 * d:3 * d]
    go = gates[:, 3 * d:4 * d]
    cs = jax.nn.sigmoid(gf) * cs + jax.nn.sigmoid(gi) * jnp.tanh(gg)
    hs = jax.nn.sigmoid(go) * jnp.tanh(cs)
    q = hs
    e = f0 * q[:, 0:1] + f1 * q[:, 1:2] + f2 * q[:, 2:3]
    m = jnp.max(e, axis=1, keepdims=True)
    ex = jnp.exp(e - m)
    alpha = ex / jnp.sum(ex, axis=1, keepdims=True)
    r0 = jnp.sum(alpha * f0, axis=1, keepdims=True)
    r1 = jnp.sum(alpha * f1, axis=1, keepdims=True)
    r2 = jnp.sum(alpha * f2, axis=1, keepdims=True)
    q_star = jnp.concatenate([q, r0, r1, r2], axis=1)
  xx = bl1_ref[...]
  for dd in range(2 * d):
    xx = xx + jnp.dot(wl1_ref[dd], q_star[:, dd:dd + 1],
                      preferred_element_type=jnp.float32)
  xx = jnp.maximum(xx, 0.0)
  o_ref[...] = jax.nn.sigmoid(
      jnp.dot(wl2_ref[...], xx, preferred_element_type=jnp.float32)
      + bl2_ref[...])


def _pad_mat(w, rows, cols):
  return jnp.zeros((rows, cols), jnp.float32).at[:w.shape[0], :w.shape[1]].set(w)


def kernel(x, edge_index, pathway_idx, W1, b1, W2, b2,
           Wih, Whh, bih, bhh, Wl1, bl1, Wl2, bl2):
  n, d = x.shape
  e = edge_index.shape[1]
  p, l = pathway_idx.shape
  nr = n * _F // 128

  ei3 = edge_index.reshape(2, e // _SUB, _SUB)
  zeros8 = jnp.zeros((n, _F), jnp.float32)
  x8 = jnp.pad(x, ((0, 0), (0, _F - d)))
  xr = x8.reshape(nr, 128)

  eye16 = jnp.eye(16, dtype=jnp.float32)
  wbig1 = jnp.kron(eye16, _pad_mat(W1.T, _F, _F))
  bbig1 = jnp.tile(_pad_mat(b1.reshape(1, -1), 1, _F), (1, 16))
  wbig2 = jnp.kron(eye16, _pad_mat(W2.T, _F, _F))
  bbig2 = jnp.tile(_pad_mat(b2.reshape(1, -1), 1, _F), (1, 16))

  agg = _edge_agg_call(n, e // _SUB)

  part1 = agg(x8, zeros8, ei3)
  h1r = _dense(xr, part1.reshape(2, nr, 128), wbig1, bbig1, 1250)
  h1 = h1r.reshape(n, _F)
  part2 = agg(h1, zeros8, ei3)
  h2r = _dense(h1r, part2.reshape(2, nr, 128), wbig2, bbig2, 1250)
  h2 = h2r.reshape(n, _F)

  pl_flat = p * l
  g_unit = _NW * _SUB
  g_pad = ((pl_flat + g_unit - 1) // g_unit) * g_unit
  pidx = jnp.concatenate(
      [pathway_idx.reshape(-1), jnp.zeros((g_pad - pl_flat,), jnp.int32)]
  ).reshape(g_pad // _SUB, _SUB)
  feat = _gather_rows_call(g_pad)(h2, pidx)
  feat = feat[:pl_flat].reshape(p, l, _F)
  f0 = feat[:, :, 0]
  f1 = feat[:, :, 1]
  f2 = feat[:, :, 2]

  wl1_stack = jnp.transpose(Wl1.reshape(p, p, 2 * d), (2, 0, 1))
  res = pl.pallas_call(
      _head_body,
      in_specs=[
          pl.BlockSpec((p, l), lambda: (0, 0)),
          pl.BlockSpec((p, l), lambda: (0, 0)),
          pl.BlockSpec((p, l), lambda: (0, 0)),
          pl.BlockSpec((2 * d, 4 * d), lambda: (0, 0)),
          pl.BlockSpec((d, 4 * d), lambda: (0, 0)),
          pl.BlockSpec((1, 4 * d), lambda: (0, 0)),
          pl.BlockSpec((1, 4 * d), lambda: (0, 0)),
          pl.BlockSpec((2 * d, p, p), lambda: (0, 0, 0)),
          pl.BlockSpec((p, 1), lambda: (0, 0)),
          pl.BlockSpec((1, p), lambda: (0, 0)),
          pl.BlockSpec((1, 1), lambda: (0, 0)),
      ],
      out_specs=pl.BlockSpec((1, 1), lambda: (0, 0)),
      out_shape=jax.ShapeDtypeStruct((1, 1), jnp.float32),
  )(f0, f1, f2, Wih.T, Whh.T, bih.reshape(1, 4 * d), bhh.reshape(1, 4 * d),
    wl1_stack, bl1.reshape(p, 1), Wl2, bl2.reshape(1, 1))
  return res.reshape(1)


# speedup vs baseline: 38.7400x; 1.0772x over previous
import functools

import jax
import jax.numpy as jnp
from jax import lax
from jax.experimental import pallas as pl
from jax.experimental.pallas import tpu as pltpu
from jax.experimental.pallas import tpu_sc as plsc

_NC = 2
_NS = 16
_NW = _NC * _NS
_SUB = 128
_K = 8
_F = 8


def _edge_agg_call(n, total_rows):
  zrows = n // _NS
  total_chunks = total_rows // _K
  nch_lo = total_chunks // _NW
  n_hi = total_chunks - nch_lo * _NW
  mesh = plsc.VectorSubcoreMesh(core_axis_name="c", subcore_axis_name="s")

  @functools.partial(
      pl.kernel,
      out_type=jax.ShapeDtypeStruct((_NC, n, _F), jnp.float32),
      mesh=mesh,
      scratch_types=[
          pltpu.VMEM((_K * _SUB,), jnp.int32),
          pltpu.VMEM((_K, _SUB), jnp.int32),
          pltpu.VMEM((_K * _SUB, _F), jnp.float32),
          pltpu.VMEM_SHARED((n, _F), jnp.float32),
          pltpu.SemaphoreType.DMA,
      ],
      compiler_params=pltpu.CompilerParams(use_tc_tiling_on_sc=False),
  )
  def k(table_hbm, zero_hbm, src_hbm, dst_hbm, out_hbm,
        idx_s, idx_d, rows, agg_sp, sem):
    c = lax.axis_index("c")
    s = lax.axis_index("s")
    wid = c * _NS + s
    pltpu.sync_copy(zero_hbm.at[pl.ds(s * zrows, zrows)],
                    agg_sp.at[pl.ds(s * zrows, zrows)])
    plsc.subcore_barrier()

    base_chunk = wid * nch_lo + jnp.minimum(wid, n_hi)
    nch = nch_lo + jnp.where(wid < n_hi, 1, 0)

    def body(i, carry):
      base = (base_chunk + i) * _K
      pltpu.sync_copy(src_hbm.at[pl.ds(base * _SUB, _K * _SUB)], idx_s)
      pltpu.sync_copy(dst_hbm.at[pl.ds(base, _K)], idx_d)
      cps = []
      for j in range(_K):
        cps.append(pltpu.async_copy(
            table_hbm.at[idx_s.at[pl.ds(j * _SUB, _SUB)]],
            rows.at[pl.ds(j * _SUB, _SUB)], sem))
      for cp in cps:
        cp.wait()
      for j in range(_K):
        pltpu.sync_copy(rows.at[pl.ds(j * _SUB, _SUB)],
                        agg_sp.at[idx_d.at[j]], add=True)
      return carry

    lax.fori_loop(0, nch, body, 0)
    plsc.subcore_barrier()
    pltpu.sync_copy(agg_sp.at[pl.ds(s * zrows, zrows)],
                    out_hbm.at[c, pl.ds(s * zrows, zrows)])

  return k


def _gather_rows_call(n_rows_out):
  rows_w = n_rows_out // (_NW * _SUB)
  per_w = rows_w * _SUB
  mesh = plsc.VectorSubcoreMesh(core_axis_name="c", subcore_axis_name="s")

  @functools.partial(
      pl.kernel,
      out_type=jax.ShapeDtypeStruct((n_rows_out, _F), jnp.float32),
      mesh=mesh,
      scratch_types=[
          pltpu.VMEM((rows_w, _SUB), jnp.int32),
          pltpu.VMEM((per_w, _F), jnp.float32),
          pltpu.SemaphoreType.DMA,
      ],
      compiler_params=pltpu.CompilerParams(use_tc_tiling_on_sc=False),
  )
  def k(table_hbm, idx_hbm, out_hbm, idx_v, rows, sem):
    c = lax.axis_index("c")
    s = lax.axis_index("s")
    wid = c * _NS + s
    pltpu.sync_copy(idx_hbm.at[pl.ds(wid * rows_w, rows_w)], idx_v)
    cps = []
    for j in range(rows_w):
      cps.append(pltpu.async_copy(
          table_hbm.at[idx_v.at[j]],
          rows.at[pl.ds(j * _SUB, _SUB)], sem))
    for cp in cps:
      cp.wait()
    pltpu.sync_copy(rows, out_hbm.at[pl.ds(wid * per_w, per_w)])

  return k


def _dense_body(x_ref, p_ref, w_ref, b_ref, o_ref):
  a = x_ref[...] + p_ref[0] + p_ref[1]
  o_ref[...] = jnp.maximum(
      jnp.dot(a, w_ref[...], preferred_element_type=jnp.float32) + b_ref[...],
      0.0)


def _dense(xr, pr, wbig, bbig, blk):
  nr = xr.shape[0]
  del blk
  return pl.pallas_call(
      _dense_body,
      in_specs=[
          pl.BlockSpec((nr, 128), lambda: (0, 0)),
          pl.BlockSpec((2, nr, 128), lambda: (0, 0, 0)),
          pl.BlockSpec((128, 128), lambda: (0, 0)),
          pl.BlockSpec((1, 128), lambda: (0, 0)),
      ],
      out_specs=pl.BlockSpec((nr, 128), lambda: (0, 0)),
      out_shape=jax.ShapeDtypeStruct((nr, 128), jnp.float32),
  )(xr, pr, wbig, bbig)


def _head_body(feat_ref, sel_ref, wih_ref, whh_ref, bih_ref, bhh_ref,
               wl1_ref, bl1_ref, wl2_ref, bl2_ref, o_ref):
  p = feat_ref.shape[0]
  d = 3
  featv = feat_ref[...]
  f0 = jnp.dot(featv, sel_ref[0], preferred_element_type=jnp.float32)
  f1 = jnp.dot(featv, sel_ref[1], preferred_element_type=jnp.float32)
  f2 = jnp.dot(featv, sel_ref[2], preferred_element_type=jnp.float32)
  wih_t = wih_ref[...]
  whh_t = whh_ref[...]
  bsum = bih_ref[...] + bhh_ref[...]
  q_star = jnp.zeros((p, 2 * d), jnp.float32)
  hs = jnp.zeros((p, d), jnp.float32)
  cs = jnp.zeros((p, d), jnp.float32)
  for _ in range(2):
    gates = (jnp.dot(q_star, wih_t, preferred_element_type=jnp.float32)
             + jnp.dot(hs, whh_t, preferred_element_type=jnp.float32) + bsum)
    gi = gates[:, 0:d]
    gf = gates[:, d:2 * d]
    gg = gates[:, 2 * d:3 * d]
    go = gates[:, 3 * d:4 * d]
    cs = jax.nn.sigmoid(gf) * cs + jax.nn.sigmoid(gi) * jnp.tanh(gg)
    hs = jax.nn.sigmoid(go) * jnp.tanh(cs)
    q = hs
    e = f0 * q[:, 0:1] + f1 * q[:, 1:2] + f2 * q[:, 2:3]
    m = jnp.max(e, axis=1, keepdims=True)
    ex = jnp.exp(e - m)
    alpha = ex / jnp.sum(ex, axis=1, keepdims=True)
    r0 = jnp.sum(alpha * f0, axis=1, keepdims=True)
    r1 = jnp.sum(alpha * f1, axis=1, keepdims=True)
    r2 = jnp.sum(alpha * f2, axis=1, keepdims=True)
    q_star = jnp.concatenate([q, r0, r1, r2], axis=1)
  xx = bl1_ref[...]
  for dd in range(2 * d):
    xx = xx + jnp.dot(wl1_ref[dd], q_star[:, dd:dd + 1],
                      preferred_element_type=jnp.float32)
  xx = jnp.maximum(xx, 0.0)
  o_ref[...] = jax.nn.sigmoid(
      jnp.dot(wl2_ref[...], xx, preferred_element_type=jnp.float32)
      + bl2_ref[...])


def _pad_mat(w, rows, cols):
  return jnp.zeros((rows, cols), jnp.float32).at[:w.shape[0], :w.shape[1]].set(w)


def kernel(x, edge_index, pathway_idx, W1, b1, W2, b2,
           Wih, Whh, bih, bhh, Wl1, bl1, Wl2, bl2):
  n, d = x.shape
  e = edge_index.shape[1]
  p, l = pathway_idx.shape
  nr = n * _F // 128

  src1 = edge_index[0]
  dst3 = edge_index[1].reshape(e // _SUB, _SUB)
  zeros8 = jnp.zeros((n, _F), jnp.float32)
  x8 = jnp.pad(x, ((0, 0), (0, _F - d)))
  xr = x8.reshape(nr, 128)

  eye16 = jnp.eye(16, dtype=jnp.float32)
  wbig1 = jnp.kron(eye16, _pad_mat(W1.T, _F, _F))
  bbig1 = jnp.tile(_pad_mat(b1.reshape(1, -1), 1, _F), (1, 16))
  wbig2 = jnp.kron(eye16, _pad_mat(W2.T, _F, _F))
  bbig2 = jnp.tile(_pad_mat(b2.reshape(1, -1), 1, _F), (1, 16))

  agg = _edge_agg_call(n, e // _SUB)

  part1 = agg(x8, zeros8, src1, dst3)
  h1r = _dense(xr, part1.reshape(2, nr, 128), wbig1, bbig1, 1250)
  h1 = h1r.reshape(n, _F)
  part2 = agg(h1, zeros8, src1, dst3)
  h2r = _dense(h1r, part2.reshape(2, nr, 128), wbig2, bbig2, 1250)
  h2 = h2r.reshape(n, _F)

  pl_flat = p * l
  g_unit = _NW * _SUB
  g_pad = ((pl_flat + g_unit - 1) // g_unit) * g_unit
  pidx = jnp.concatenate(
      [pathway_idx.reshape(-1), jnp.zeros((g_pad - pl_flat,), jnp.int32)]
  ).reshape(g_pad // _SUB, _SUB)
  feat2 = _gather_rows_call(g_pad)(h2, pidx)[:pl_flat].reshape(p, l * _F)

  eye_l = jnp.eye(l, dtype=jnp.float32)
  sel = jnp.stack([
      jnp.kron(eye_l, jnp.eye(_F, dtype=jnp.float32)[:, dd:dd + 1])
      for dd in range(d)])

  wl1_stack = jnp.transpose(Wl1.reshape(p, p, 2 * d), (2, 0, 1))
  res = pl.pallas_call(
      _head_body,
      in_specs=[
          pl.BlockSpec((p, l * _F), lambda: (0, 0)),
          pl.BlockSpec((d, l * _F, l), lambda: (0, 0, 0)),
          pl.BlockSpec((2 * d, 4 * d), lambda: (0, 0)),
          pl.BlockSpec((d, 4 * d), lambda: (0, 0)),
          pl.BlockSpec((1, 4 * d), lambda: (0, 0)),
          pl.BlockSpec((1, 4 * d), lambda: (0, 0)),
          pl.BlockSpec((2 * d, p, p), lambda: (0, 0, 0)),
          pl.BlockSpec((p, 1), lambda: (0, 0)),
          pl.BlockSpec((1, p), lambda: (0, 0)),
          pl.BlockSpec((1, 1), lambda: (0, 0)),
      ],
      out_specs=pl.BlockSpec((1, 1), lambda: (0, 0)),
      out_shape=jax.ShapeDtypeStruct((1, 1), jnp.float32),
  )(feat2, sel, Wih.T, Whh.T, bih.reshape(1, 4 * d), bhh.reshape(1, 4 * d),
    wl1_stack, bl1.reshape(p, 1), Wl2, bl2.reshape(1, 1))
  return res.reshape(1)


# speedup vs baseline: 44.3770x; 1.1455x over previous
import functools

import jax
import jax.numpy as jnp
from jax import lax
from jax.experimental import pallas as pl
from jax.experimental.pallas import tpu as pltpu
from jax.experimental.pallas import tpu_sc as plsc

_NC = 2
_NS = 16
_NW = _NC * _NS
_SUB = 128
_K = 8
_F = 8


def _edge_agg_call(n, total_rows):
  zrows = n // _NS
  total_chunks = total_rows // _K
  nch_lo = total_chunks // _NW
  n_hi = total_chunks - nch_lo * _NW
  mesh = plsc.VectorSubcoreMesh(core_axis_name="c", subcore_axis_name="s")

  @functools.partial(
      pl.kernel,
      out_type=jax.ShapeDtypeStruct((_NC, n, _F), jnp.float32),
      mesh=mesh,
      scratch_types=[
          pltpu.VMEM((_K * _SUB,), jnp.int32),
          pltpu.VMEM((_K, _SUB), jnp.int32),
          pltpu.VMEM((_K * _SUB, _F), jnp.float32),
          pltpu.VMEM_SHARED((n, _F), jnp.float32),
          pltpu.SemaphoreType.DMA,
          pltpu.SemaphoreType.DMA,
      ],
      compiler_params=pltpu.CompilerParams(use_tc_tiling_on_sc=False),
  )
  def k(table_hbm, zero_hbm, src_hbm, dst_hbm, out_hbm,
        idx_s, idx_d, rows, agg_sp, sem, sem2):
    c = lax.axis_index("c")
    s = lax.axis_index("s")
    wid = c * _NS + s
    pltpu.sync_copy(zero_hbm.at[pl.ds(s * zrows, zrows)],
                    agg_sp.at[pl.ds(s * zrows, zrows)])
    plsc.subcore_barrier()

    base_chunk = wid * nch_lo + jnp.minimum(wid, n_hi)
    nch = nch_lo + jnp.where(wid < n_hi, 1, 0)

    def body(i, carry):
      base = (base_chunk + i) * _K
      pltpu.sync_copy(src_hbm.at[pl.ds(base * _SUB, _K * _SUB)], idx_s)
      pltpu.sync_copy(dst_hbm.at[pl.ds(base, _K)], idx_d)
      cps = []
      for j in range(_K):
        cps.append(pltpu.async_copy(
            table_hbm.at[idx_s.at[pl.ds(j * _SUB, _SUB)]],
            rows.at[pl.ds(j * _SUB, _SUB)], sem))
      scs = []
      for j in range(_K):
        cps[j].wait()
        scs.append(pltpu.async_copy(
            rows.at[pl.ds(j * _SUB, _SUB)],
            agg_sp.at[idx_d.at[j]], sem2, add=True))
      for cp in scs:
        cp.wait()
      return carry

    lax.fori_loop(0, nch, body, 0)
    plsc.subcore_barrier()
    pltpu.sync_copy(agg_sp.at[pl.ds(s * zrows, zrows)],
                    out_hbm.at[c, pl.ds(s * zrows, zrows)])

  return k


def _gather_rows_call(n_rows_out):
  rows_w = n_rows_out // (_NW * _SUB)
  per_w = rows_w * _SUB
  mesh = plsc.VectorSubcoreMesh(core_axis_name="c", subcore_axis_name="s")

  @functools.partial(
      pl.kernel,
      out_type=jax.ShapeDtypeStruct((n_rows_out, _F), jnp.float32),
      mesh=mesh,
      scratch_types=[
          pltpu.VMEM((rows_w, _SUB), jnp.int32),
          pltpu.VMEM((per_w, _F), jnp.float32),
          pltpu.SemaphoreType.DMA,
      ],
      compiler_params=pltpu.CompilerParams(use_tc_tiling_on_sc=False),
  )
  def k(table_hbm, idx_hbm, out_hbm, idx_v, rows, sem):
    c = lax.axis_index("c")
    s = lax.axis_index("s")
    wid = c * _NS + s
    pltpu.sync_copy(idx_hbm.at[pl.ds(wid * rows_w, rows_w)], idx_v)
    cps = []
    for j in range(rows_w):
      cps.append(pltpu.async_copy(
          table_hbm.at[idx_v.at[j]],
          rows.at[pl.ds(j * _SUB, _SUB)], sem))
    for cp in cps:
      cp.wait()
    pltpu.sync_copy(rows, out_hbm.at[pl.ds(wid * per_w, per_w)])

  return k


def _dense_body(x_ref, p_ref, w_ref, b_ref, o_ref):
  a = x_ref[...] + p_ref[0] + p_ref[1]
  o_ref[...] = jnp.maximum(
      jnp.dot(a, w_ref[...], preferred_element_type=jnp.float32) + b_ref[...],
      0.0)


def _dense(xr, pr, wbig, bbig, blk):
  nr = xr.shape[0]
  del blk
  return pl.pallas_call(
      _dense_body,
      in_specs=[
          pl.BlockSpec((nr, 128), lambda: (0, 0)),
          pl.BlockSpec((2, nr, 128), lambda: (0, 0, 0)),
          pl.BlockSpec((128, 128), lambda: (0, 0)),
          pl.BlockSpec((1, 128), lambda: (0, 0)),
      ],
      out_specs=pl.BlockSpec((nr, 128), lambda: (0, 0)),
      out_shape=jax.ShapeDtypeStruct((nr, 128), jnp.float32),
  )(xr, pr, wbig, bbig)


def _head_body(feat_ref, sel_ref, wih_ref, whh_ref, bih_ref, bhh_ref,
               wl1_ref, bl1_ref, wl2_ref, bl2_ref, o_ref):
  p = feat_ref.shape[0]
  d = 3
  featv = feat_ref[...]
  f0 = jnp.dot(featv, sel_ref[0], preferred_element_type=jnp.float32)
  f1 = jnp.dot(featv, sel_ref[1], preferred_element_type=jnp.float32)
  f2 = jnp.dot(featv, sel_ref[2], preferred_element_type=jnp.float32)
  wih_t = wih_ref[...]
  whh_t = whh_ref[...]
  bsum = bih_ref[...] + bhh_ref[...]
  q_star = jnp.zeros((p, 2 * d), jnp.float32)
  hs = jnp.zeros((p, d), jnp.float32)
  cs = jnp.zeros((p, d), jnp.float32)
  for _ in range(2):
    gates = (jnp.dot(q_star, wih_t, preferred_element_type=jnp.float32)
             + jnp.dot(hs, whh_t, preferred_element_type=jnp.float32) + bsum)
    gi = gates[:, 0:d]
    gf = gates[:, d:2 * d]
    gg = gates[:, 2 * d:3 * d]
    go = gates[:, 3 * d:4 * d]
    cs = jax.nn.sigmoid(gf) * cs + jax.nn.sigmoid(gi) * jnp.tanh(gg)
    hs = jax.nn.sigmoid(go) * jnp.tanh(cs)
    q = hs
    e = f0 * q[:, 0:1] + f1 * q[:, 1:2] + f2 * q[:, 2:3]
    m = jnp.max(e, axis=1, keepdims=True)
    ex = jnp.exp(e - m)
    alpha = ex / jnp.sum(ex, axis=1, keepdims=True)
    r0 = jnp.sum(alpha * f0, axis=1, keepdims=True)
    r1 = jnp.sum(alpha * f1, axis=1, keepdims=True)
    r2 = jnp.sum(alpha * f2, axis=1, keepdims=True)
    q_star = jnp.concatenate([q, r0, r1, r2], axis=1)
  xx = bl1_ref[...]
  for dd in range(2 * d):
    xx = xx + jnp.dot(wl1_ref[dd], q_star[:, dd:dd + 1],
                      preferred_element_type=jnp.float32)
  xx = jnp.maximum(xx, 0.0)
  o_ref[...] = jax.nn.sigmoid(
      jnp.dot(wl2_ref[...], xx, preferred_element_type=jnp.float32)
      + bl2_ref[...])


def _pad_mat(w, rows, cols):
  return jnp.zeros((rows, cols), jnp.float32).at[:w.shape[0], :w.shape[1]].set(w)


def kernel(x, edge_index, pathway_idx, W1, b1, W2, b2,
           Wih, Whh, bih, bhh, Wl1, bl1, Wl2, bl2):
  n, d = x.shape
  e = edge_index.shape[1]
  p, l = pathway_idx.shape
  nr = n * _F // 128

  src1 = edge_index[0]
  dst3 = edge_index[1].reshape(e // _SUB, _SUB)
  zeros8 = jnp.zeros((n, _F), jnp.float32)
  x8 = jnp.pad(x, ((0, 0), (0, _F - d)))
  xr = x8.reshape(nr, 128)

  eye16 = jnp.eye(16, dtype=jnp.float32)
  wbig1 = jnp.kron(eye16, _pad_mat(W1.T, _F, _F))
  bbig1 = jnp.tile(_pad_mat(b1.reshape(1, -1), 1, _F), (1, 16))
  wbig2 = jnp.kron(eye16, _pad_mat(W2.T, _F, _F))
  bbig2 = jnp.tile(_pad_mat(b2.reshape(1, -1), 1, _F), (1, 16))

  agg = _edge_agg_call(n, e // _SUB)

  part1 = agg(x8, zeros8, src1, dst3)
  h1r = _dense(xr, part1.reshape(2, nr, 128), wbig1, bbig1, 1250)
  h1 = h1r.reshape(n, _F)
  part2 = agg(h1, zeros8, src1, dst3)
  h2r = _dense(h1r, part2.reshape(2, nr, 128), wbig2, bbig2, 1250)
  h2 = h2r.reshape(n, _F)

  pl_flat = p * l
  g_unit = _NW * _SUB
  g_pad = ((pl_flat + g_unit - 1) // g_unit) * g_unit
  pidx = jnp.concatenate(
      [pathway_idx.reshape(-1), jnp.zeros((g_pad - pl_flat,), jnp.int32)]
  ).reshape(g_pad // _SUB, _SUB)
  feat2 = _gather_rows_call(g_pad)(h2, pidx)[:pl_flat].reshape(p, l * _F)

  eye_l = jnp.eye(l, dtype=jnp.float32)
  sel = jnp.stack([
      jnp.kron(eye_l, jnp.eye(_F, dtype=jnp.float32)[:, dd:dd + 1])
      for dd in range(d)])

  wl1_stack = jnp.transpose(Wl1.reshape(p, p, 2 * d), (2, 0, 1))
  res = pl.pallas_call(
      _head_body,
      in_specs=[
          pl.BlockSpec((p, l * _F), lambda: (0, 0)),
          pl.BlockSpec((d, l * _F, l), lambda: (0, 0, 0)),
          pl.BlockSpec((2 * d, 4 * d), lambda: (0, 0)),
          pl.BlockSpec((d, 4 * d), lambda: (0, 0)),
          pl.BlockSpec((1, 4 * d), lambda: (0, 0)),
          pl.BlockSpec((1, 4 * d), lambda: (0, 0)),
          pl.BlockSpec((2 * d, p, p), lambda: (0, 0, 0)),
          pl.BlockSpec((p, 1), lambda: (0, 0)),
          pl.BlockSpec((1, p), lambda: (0, 0)),
          pl.BlockSpec((1, 1), lambda: (0, 0)),
      ],
      out_specs=pl.BlockSpec((1, 1), lambda: (0, 0)),
      out_shape=jax.ShapeDtypeStruct((1, 1), jnp.float32),
  )(feat2, sel, Wih.T, Whh.T, bih.reshape(1, 4 * d), bhh.reshape(1, 4 * d),
    wl1_stack, bl1.reshape(p, 1), Wl2, bl2.reshape(1, 1))
  return res.reshape(1)


# speedup vs baseline: 54.5831x; 1.2300x over previous
import functools

import jax
import jax.numpy as jnp
from jax import lax
from jax.experimental import pallas as pl
from jax.experimental.pallas import tpu as pltpu
from jax.experimental.pallas import tpu_sc as plsc

_NC = 2
_NS = 16
_NW = _NC * _NS
_SUB = 128
_K = 16
_F = 8


def _edge_agg_call(n, total_rows):
  zrows = n // _NS
  total_chunks = total_rows // _K
  nch_lo = total_chunks // _NW
  n_hi = total_chunks - nch_lo * _NW
  mesh = plsc.VectorSubcoreMesh(core_axis_name="c", subcore_axis_name="s")

  @functools.partial(
      pl.kernel,
      out_type=jax.ShapeDtypeStruct((_NC, n, _F), jnp.float32),
      mesh=mesh,
      scratch_types=[
          pltpu.VMEM((_K * _SUB,), jnp.int32),
          pltpu.VMEM((_K, _SUB), jnp.int32),
          pltpu.VMEM((_K * _SUB, _F), jnp.float32),
          pltpu.VMEM_SHARED((n, _F), jnp.float32),
          pltpu.SemaphoreType.DMA,
          pltpu.SemaphoreType.DMA,
      ],
      compiler_params=pltpu.CompilerParams(use_tc_tiling_on_sc=False),
  )
  def k(table_hbm, zero_hbm, src_hbm, dst_hbm, out_hbm,
        idx_s, idx_d, rows, agg_sp, sem, sem2):
    c = lax.axis_index("c")
    s = lax.axis_index("s")
    wid = c * _NS + s
    pltpu.sync_copy(zero_hbm.at[pl.ds(s * zrows, zrows)],
                    agg_sp.at[pl.ds(s * zrows, zrows)])
    plsc.subcore_barrier()

    base_chunk = wid * nch_lo + jnp.minimum(wid, n_hi)
    nch = nch_lo + jnp.where(wid < n_hi, 1, 0)

    def body(i, carry):
      base = (base_chunk + i) * _K
      pltpu.sync_copy(src_hbm.at[pl.ds(base * _SUB, _K * _SUB)], idx_s)
      pltpu.sync_copy(dst_hbm.at[pl.ds(base, _K)], idx_d)
      cps = []
      for j in range(_K):
        cps.append(pltpu.async_copy(
            table_hbm.at[idx_s.at[pl.ds(j * _SUB, _SUB)]],
            rows.at[pl.ds(j * _SUB, _SUB)], sem))
      scs = []
      for j in range(_K):
        cps[j].wait()
        scs.append(pltpu.async_copy(
            rows.at[pl.ds(j * _SUB, _SUB)],
            agg_sp.at[idx_d.at[j]], sem2, add=True))
      for cp in scs:
        cp.wait()
      return carry

    lax.fori_loop(0, nch, body, 0)
    plsc.subcore_barrier()
    pltpu.sync_copy(agg_sp.at[pl.ds(s * zrows, zrows)],
                    out_hbm.at[c, pl.ds(s * zrows, zrows)])

  return k


def _gather_rows_call(n_rows_out):
  rows_w = n_rows_out // (_NW * _SUB)
  per_w = rows_w * _SUB
  mesh = plsc.VectorSubcoreMesh(core_axis_name="c", subcore_axis_name="s")

  @functools.partial(
      pl.kernel,
      out_type=jax.ShapeDtypeStruct((n_rows_out, _F), jnp.float32),
      mesh=mesh,
      scratch_types=[
          pltpu.VMEM((rows_w, _SUB), jnp.int32),
          pltpu.VMEM((per_w, _F), jnp.float32),
          pltpu.SemaphoreType.DMA,
      ],
      compiler_params=pltpu.CompilerParams(use_tc_tiling_on_sc=False),
  )
  def k(table_hbm, idx_hbm, out_hbm, idx_v, rows, sem):
    c = lax.axis_index("c")
    s = lax.axis_index("s")
    wid = c * _NS + s
    pltpu.sync_copy(idx_hbm.at[pl.ds(wid * rows_w, rows_w)], idx_v)
    cps = []
    for j in range(rows_w):
      cps.append(pltpu.async_copy(
          table_hbm.at[idx_v.at[j]],
          rows.at[pl.ds(j * _SUB, _SUB)], sem))
    for cp in cps:
      cp.wait()
    pltpu.sync_copy(rows, out_hbm.at[pl.ds(wid * per_w, per_w)])

  return k


def _dense_body(x_ref, p_ref, w_ref, b_ref, o_ref):
  a = x_ref[...] + p_ref[0] + p_ref[1]
  o_ref[...] = jnp.maximum(
      jnp.dot(a, w_ref[...], preferred_element_type=jnp.float32) + b_ref[...],
      0.0)


def _dense(xr, pr, wbig, bbig, blk):
  nr = xr.shape[0]
  del blk
  return pl.pallas_call(
      _dense_body,
      in_specs=[
          pl.BlockSpec((nr, 128), lambda: (0, 0)),
          pl.BlockSpec((2, nr, 128), lambda: (0, 0, 0)),
          pl.BlockSpec((128, 128), lambda: (0, 0)),
          pl.BlockSpec((1, 128), lambda: (0, 0)),
      ],
      out_specs=pl.BlockSpec((nr, 128), lambda: (0, 0)),
      out_shape=jax.ShapeDtypeStruct((nr, 128), jnp.float32),
  )(xr, pr, wbig, bbig)


def _head_body(feat_ref, sel_ref, wih_ref, whh_ref, bih_ref, bhh_ref,
               wl1_ref, bl1_ref, wl2_ref, bl2_ref, o_ref):
  p = feat_ref.shape[0]
  d = 3
  featv = feat_ref[...]
  f0 = jnp.dot(featv, sel_ref[0], preferred_element_type=jnp.float32)
  f1 = jnp.dot(featv, sel_ref[1], preferred_element_type=jnp.float32)
  f2 = jnp.dot(featv, sel_ref[2], preferred_element_type=jnp.float32)
  wih_t = wih_ref[...]
  whh_t = whh_ref[...]
  bsum = bih_ref[...] + bhh_ref[...]
  q_star = jnp.zeros((p, 2 * d), jnp.float32)
  hs = jnp.zeros((p, d), jnp.float32)
  cs = jnp.zeros((p, d), jnp.float32)
  for _ in range(2):
    gates = (jnp.dot(q_star, wih_t, preferred_element_type=jnp.float32)
             + jnp.dot(hs, whh_t, preferred_element_type=jnp.float32) + bsum)
    gi = gates[:, 0:d]
    gf = gates[:, d:2 * d]
    gg = gates[:, 2 * d:3 * d]
    go = gates[:, 3 * d:4 * d]
    cs = jax.nn.sigmoid(gf) * cs + jax.nn.sigmoid(gi) * jnp.tanh(gg)
    hs = jax.nn.sigmoid(go) * jnp.tanh(cs)
    q = hs
    e = f0 * q[:, 0:1] + f1 * q[:, 1:2] + f2 * q[:, 2:3]
    m = jnp.max(e, axis=1, keepdims=True)
    ex = jnp.exp(e - m)
    alpha = ex / jnp.sum(ex, axis=1, keepdims=True)
    r0 = jnp.sum(alpha * f0, axis=1, keepdims=True)
    r1 = jnp.sum(alpha * f1, axis=1, keepdims=True)
    r2 = jnp.sum(alpha * f2, axis=1, keepdims=True)
    q_star = jnp.concatenate([q, r0, r1, r2], axis=1)
  xx = bl1_ref[...]
  for dd in range(2 * d):
    xx = xx + jnp.dot(wl1_ref[dd], q_star[:, dd:dd + 1],
                      preferred_element_type=jnp.float32)
  xx = jnp.maximum(xx, 0.0)
  o_ref[...] = jax.nn.sigmoid(
      jnp.dot(wl2_ref[...], xx, preferred_element_type=jnp.float32)
      + bl2_ref[...])


def _pad_mat(w, rows, cols):
  return jnp.zeros((rows, cols), jnp.float32).at[:w.shape[0], :w.shape[1]].set(w)


def kernel(x, edge_index, pathway_idx, W1, b1, W2, b2,
           Wih, Whh, bih, bhh, Wl1, bl1, Wl2, bl2):
  n, d = x.shape
  e = edge_index.shape[1]
  p, l = pathway_idx.shape
  nr = n * _F // 128

  src1 = edge_index[0]
  dst3 = edge_index[1].reshape(e // _SUB, _SUB)
  zeros8 = jnp.zeros((n, _F), jnp.float32)
  x8 = jnp.pad(x, ((0, 0), (0, _F - d)))
  xr = x8.reshape(nr, 128)

  eye16 = jnp.eye(16, dtype=jnp.float32)
  wbig1 = jnp.kron(eye16, _pad_mat(W1.T, _F, _F))
  bbig1 = jnp.tile(_pad_mat(b1.reshape(1, -1), 1, _F), (1, 16))
  wbig2 = jnp.kron(eye16, _pad_mat(W2.T, _F, _F))
  bbig2 = jnp.tile(_pad_mat(b2.reshape(1, -1), 1, _F), (1, 16))

  agg = _edge_agg_call(n, e // _SUB)

  part1 = agg(x8, zeros8, src1, dst3)
  h1r = _dense(xr, part1.reshape(2, nr, 128), wbig1, bbig1, 1250)
  h1 = h1r.reshape(n, _F)
  part2 = agg(h1, zeros8, src1, dst3)
  h2r = _dense(h1r, part2.reshape(2, nr, 128), wbig2, bbig2, 1250)
  h2 = h2r.reshape(n, _F)

  pl_flat = p * l
  g_unit = _NW * _SUB
  g_pad = ((pl_flat + g_unit - 1) // g_unit) * g_unit
  pidx = jnp.concatenate(
      [pathway_idx.reshape(-1), jnp.zeros((g_pad - pl_flat,), jnp.int32)]
  ).reshape(g_pad // _SUB, _SUB)
  feat2 = _gather_rows_call(g_pad)(h2, pidx)[:pl_flat].reshape(p, l * _F)

  eye_l = jnp.eye(l, dtype=jnp.float32)
  sel = jnp.stack([
      jnp.kron(eye_l, jnp.eye(_F, dtype=jnp.float32)[:, dd:dd + 1])
      for dd in range(d)])

  wl1_stack = jnp.transpose(Wl1.reshape(p, p, 2 * d), (2, 0, 1))
  res = pl.pallas_call(
      _head_body,
      in_specs=[
          pl.BlockSpec((p, l * _F), lambda: (0, 0)),
          pl.BlockSpec((d, l * _F, l), lambda: (0, 0, 0)),
          pl.BlockSpec((2 * d, 4 * d), lambda: (0, 0)),
          pl.BlockSpec((d, 4 * d), lambda: (0, 0)),
          pl.BlockSpec((1, 4 * d), lambda: (0, 0)),
          pl.BlockSpec((1, 4 * d), lambda: (0, 0)),
          pl.BlockSpec((2 * d, p, p), lambda: (0, 0, 0)),
          pl.BlockSpec((p, 1), lambda: (0, 0)),
          pl.BlockSpec((1, p), lambda: (0, 0)),
          pl.BlockSpec((1, 1), lambda: (0, 0)),
      ],
      out_specs=pl.BlockSpec((1, 1), lambda: (0, 0)),
      out_shape=jax.ShapeDtypeStruct((1, 1), jnp.float32),
  )(feat2, sel, Wih.T, Whh.T, bih.reshape(1, 4 * d), bhh.reshape(1, 4 * d),
    wl1_stack, bl1.reshape(p, 1), Wl2, bl2.reshape(1, 1))
  return res.reshape(1)


# speedup vs baseline: 55.7931x; 1.0222x over previous
import functools

import jax
import jax.numpy as jnp
from jax import lax
from jax.experimental import pallas as pl
from jax.experimental.pallas import tpu as pltpu
from jax.experimental.pallas import tpu_sc as plsc

_NC = 2
_NS = 16
_NW = _NC * _NS
_SUB = 128
_K = 16
_F = 8


def _edge_agg_call(n, total_rows):
  zrows = n // _NS
  total_chunks = total_rows // _K
  nch_lo = total_chunks // _NW
  n_hi = total_chunks - nch_lo * _NW
  mesh = plsc.VectorSubcoreMesh(core_axis_name="c", subcore_axis_name="s")

  @functools.partial(
      pl.kernel,
      out_type=jax.ShapeDtypeStruct((_NC, n, _F), jnp.float32),
      mesh=mesh,
      scratch_types=[
          pltpu.VMEM((_K * _SUB,), jnp.int32),
          pltpu.VMEM((_K * _SUB,), jnp.int32),
          pltpu.VMEM((_K * _SUB, _F), jnp.float32),
          pltpu.VMEM_SHARED((n, _F), jnp.float32),
          pltpu.SemaphoreType.DMA,
          pltpu.SemaphoreType.DMA,
      ],
      compiler_params=pltpu.CompilerParams(use_tc_tiling_on_sc=False),
  )
  def k(table_hbm, zero_hbm, ei_hbm, out_hbm,
        idx_s, idx_d, rows, agg_sp, sem, sem2):
    c = lax.axis_index("c")
    s = lax.axis_index("s")
    wid = c * _NS + s
    pltpu.sync_copy(zero_hbm.at[pl.ds(s * zrows, zrows)],
                    agg_sp.at[pl.ds(s * zrows, zrows)])
    plsc.subcore_barrier()

    base_chunk = wid * nch_lo + jnp.minimum(wid, n_hi)
    nch = nch_lo + jnp.where(wid < n_hi, 1, 0)

    def body(i, carry):
      base = (base_chunk + i) * _K * _SUB
      pltpu.sync_copy(ei_hbm.at[0, pl.ds(base, _K * _SUB)], idx_s)
      pltpu.sync_copy(ei_hbm.at[1, pl.ds(base, _K * _SUB)], idx_d)
      cps = []
      for j in range(_K):
        cps.append(pltpu.async_copy(
            table_hbm.at[idx_s.at[pl.ds(j * _SUB, _SUB)]],
            rows.at[pl.ds(j * _SUB, _SUB)], sem))
      scs = []
      for j in range(_K):
        cps[j].wait()
        scs.append(pltpu.async_copy(
            rows.at[pl.ds(j * _SUB, _SUB)],
            agg_sp.at[idx_d.at[pl.ds(j * _SUB, _SUB)]], sem2, add=True))
      for cp in scs:
        cp.wait()
      return carry

    lax.fori_loop(0, nch, body, 0)
    plsc.subcore_barrier()
    pltpu.sync_copy(agg_sp.at[pl.ds(s * zrows, zrows)],
                    out_hbm.at[c, pl.ds(s * zrows, zrows)])

  return k


def _gather_rows_call(n_rows_out):
  rows_w = n_rows_out // (_NW * _SUB)
  per_w = rows_w * _SUB
  mesh = plsc.VectorSubcoreMesh(core_axis_name="c", subcore_axis_name="s")

  @functools.partial(
      pl.kernel,
      out_type=jax.ShapeDtypeStruct((n_rows_out, _F), jnp.float32),
      mesh=mesh,
      scratch_types=[
          pltpu.VMEM((rows_w, _SUB), jnp.int32),
          pltpu.VMEM((per_w, _F), jnp.float32),
          pltpu.SemaphoreType.DMA,
      ],
      compiler_params=pltpu.CompilerParams(use_tc_tiling_on_sc=False),
  )
  def k(table_hbm, idx_hbm, out_hbm, idx_v, rows, sem):
    c = lax.axis_index("c")
    s = lax.axis_index("s")
    wid = c * _NS + s
    pltpu.sync_copy(idx_hbm.at[pl.ds(wid * rows_w, rows_w)], idx_v)
    cps = []
    for j in range(rows_w):
      cps.append(pltpu.async_copy(
          table_hbm.at[idx_v.at[j]],
          rows.at[pl.ds(j * _SUB, _SUB)], sem))
    for cp in cps:
      cp.wait()
    pltpu.sync_copy(rows, out_hbm.at[pl.ds(wid * per_w, per_w)])

  return k


def _dense_body(x_ref, p_ref, w_ref, b_ref, o_ref):
  a = x_ref[...] + p_ref[0] + p_ref[1]
  o_ref[...] = jnp.maximum(
      jnp.dot(a, w_ref[...], preferred_element_type=jnp.float32) + b_ref[...],
      0.0)


def _dense(xr, pr, wbig, bbig, blk):
  nr = xr.shape[0]
  del blk
  return pl.pallas_call(
      _dense_body,
      in_specs=[
          pl.BlockSpec((nr, 128), lambda: (0, 0)),
          pl.BlockSpec((2, nr, 128), lambda: (0, 0, 0)),
          pl.BlockSpec((128, 128), lambda: (0, 0)),
          pl.BlockSpec((1, 128), lambda: (0, 0)),
      ],
      out_specs=pl.BlockSpec((nr, 128), lambda: (0, 0)),
      out_shape=jax.ShapeDtypeStruct((nr, 128), jnp.float32),
  )(xr, pr, wbig, bbig)


def _head_body(feat_ref, sel_ref, wih_ref, whh_ref, bih_ref, bhh_ref,
               wl1_ref, bl1_ref, wl2_ref, bl2_ref, o_ref):
  p = feat_ref.shape[0]
  d = 3
  featv = feat_ref[...]
  f0 = jnp.dot(featv, sel_ref[0], preferred_element_type=jnp.float32)
  f1 = jnp.dot(featv, sel_ref[1], preferred_element_type=jnp.float32)
  f2 = jnp.dot(featv, sel_ref[2], preferred_element_type=jnp.float32)
  wih_t = wih_ref[...]
  whh_t = whh_ref[...]
  bsum = bih_ref[...] + bhh_ref[...]
  q_star = jnp.zeros((p, 2 * d), jnp.float32)
  hs = jnp.zeros((p, d), jnp.float32)
  cs = jnp.zeros((p, d), jnp.float32)
  for _ in range(2):
    gates = (jnp.dot(q_star, wih_t, preferred_element_type=jnp.float32)
             + jnp.dot(hs, whh_t, preferred_element_type=jnp.float32) + bsum)
    gi = gates[:, 0:d]
    gf = gates[:, d:2 * d]
    gg = gates[:, 2 * d:3 * d]
    go = gates[:, 3 * d:4 * d]
    cs = jax.nn.sigmoid(gf) * cs + jax.nn.sigmoid(gi) * jnp.tanh(gg)
    hs = jax.nn.sigmoid(go) * jnp.tanh(cs)
    q = hs
    e = f0 * q[:, 0:1] + f1 * q[:, 1:2] + f2 * q[:, 2:3]
    m = jnp.max(e, axis=1, keepdims=True)
    ex = jnp.exp(e - m)
    alpha = ex / jnp.sum(ex, axis=1, keepdims=True)
    r0 = jnp.sum(alpha * f0, axis=1, keepdims=True)
    r1 = jnp.sum(alpha * f1, axis=1, keepdims=True)
    r2 = jnp.sum(alpha * f2, axis=1, keepdims=True)
    q_star = jnp.concatenate([q, r0, r1, r2], axis=1)
  xx = bl1_ref[...]
  for dd in range(2 * d):
    xx = xx + jnp.dot(wl1_ref[dd], q_star[:, dd:dd + 1],
                      preferred_element_type=jnp.float32)
  xx = jnp.maximum(xx, 0.0)
  o_ref[...] = jax.nn.sigmoid(
      jnp.dot(wl2_ref[...], xx, preferred_element_type=jnp.float32)
      + bl2_ref[...])


def _pad_mat(w, rows, cols):
  return jnp.zeros((rows, cols), jnp.float32).at[:w.shape[0], :w.shape[1]].set(w)


def kernel(x, edge_index, pathway_idx, W1, b1, W2, b2,
           Wih, Whh, bih, bhh, Wl1, bl1, Wl2, bl2):
  n, d = x.shape
  e = edge_index.shape[1]
  p, l = pathway_idx.shape
  nr = n * _F // 128

  zeros8 = jnp.zeros((n, _F), jnp.float32)
  x8 = jnp.pad(x, ((0, 0), (0, _F - d)))
  xr = x8.reshape(nr, 128)

  eye16 = jnp.eye(16, dtype=jnp.float32)
  wbig1 = jnp.kron(eye16, _pad_mat(W1.T, _F, _F))
  bbig1 = jnp.tile(_pad_mat(b1.reshape(1, -1), 1, _F), (1, 16))
  wbig2 = jnp.kron(eye16, _pad_mat(W2.T, _F, _F))
  bbig2 = jnp.tile(_pad_mat(b2.reshape(1, -1), 1, _F), (1, 16))

  agg = _edge_agg_call(n, e // _SUB)

  part1 = agg(x8, zeros8, edge_index)
  h1r = _dense(xr, part1.reshape(2, nr, 128), wbig1, bbig1, 1250)
  h1 = h1r.reshape(n, _F)
  part2 = agg(h1, zeros8, edge_index)
  h2r = _dense(h1r, part2.reshape(2, nr, 128), wbig2, bbig2, 1250)
  h2 = h2r.reshape(n, _F)

  pl_flat = p * l
  g_unit = _NW * _SUB
  g_pad = ((pl_flat + g_unit - 1) // g_unit) * g_unit
  pidx = jnp.concatenate(
      [pathway_idx.reshape(-1), jnp.zeros((g_pad - pl_flat,), jnp.int32)]
  ).reshape(g_pad // _SUB, _SUB)
  feat2 = _gather_rows_call(g_pad)(h2, pidx)[:pl_flat].reshape(p, l * _F)

  eye_l = jnp.eye(l, dtype=jnp.float32)
  sel = jnp.stack([
      jnp.kron(eye_l, jnp.eye(_F, dtype=jnp.float32)[:, dd:dd + 1])
      for dd in range(d)])

  wl1_stack = jnp.transpose(Wl1.reshape(p, p, 2 * d), (2, 0, 1))
  res = pl.pallas_call(
      _head_body,
      in_specs=[
          pl.BlockSpec((p, l * _F), lambda: (0, 0)),
          pl.BlockSpec((d, l * _F, l), lambda: (0, 0, 0)),
          pl.BlockSpec((2 * d, 4 * d), lambda: (0, 0)),
          pl.BlockSpec((d, 4 * d), lambda: (0, 0)),
          pl.BlockSpec((1, 4 * d), lambda: (0, 0)),
          pl.BlockSpec((1, 4 * d), lambda: (0, 0)),
          pl.BlockSpec((2 * d, p, p), lambda: (0, 0, 0)),
          pl.BlockSpec((p, 1), lambda: (0, 0)),
          pl.BlockSpec((1, p), lambda: (0, 0)),
          pl.BlockSpec((1, 1), lambda: (0, 0)),
      ],
      out_specs=pl.BlockSpec((1, 1), lambda: (0, 0)),
      out_shape=jax.ShapeDtypeStruct((1, 1), jnp.float32),
  )(feat2, sel, Wih.T, Whh.T, bih.reshape(1, 4 * d), bhh.reshape(1, 4 * d),
    wl1_stack, bl1.reshape(p, 1), Wl2, bl2.reshape(1, 1))
  return res.reshape(1)
